# baseline (speedup 1.0000x reference)
import functools

import jax
import jax.numpy as jnp
from jax import lax
from jax.experimental import pallas as pl
from jax.experimental.pallas import tpu as pltpu
from jax.experimental.pallas import tpu_sc as plsc

_B = 4
_E = 8
_K = 2
_L = 16
_NC = 2
_NS = 16
_NW = _NC * _NS

_N = 4 * 1024 * 1024
_SC_FRAC_NUM = 16
_SC_CHUNK = 4096
_TC_CHUNK = 256 * 1024


def _sc_body(x_hbm, w_hbm, out_hbm, x_v, w_v, acc_v, *, n_w, nchunks):
    wid = lax.axis_index("s") * _NC + lax.axis_index("c")
    base = wid * n_w

    def chunk_body(ci, accs):
        off = base + ci * _SC_CHUNK
        pltpu.sync_copy(x_hbm.at[:, pl.ds(off, _SC_CHUNK)], x_v)
        pltpu.sync_copy(w_hbm.at[:, pl.ds(off, _SC_CHUNK)], w_v)

        def vec_body(i, accs):
            xs = [x_v[b, pl.ds(i * _L, _L)] for b in range(_B)]
            ws = [w_v[e, pl.ds(i * _L, _L)] for e in range(_E)]
            return tuple(accs[b * _E + e] + xs[b] * ws[e]
                         for b in range(_B) for e in range(_E))

        return lax.fori_loop(0, _SC_CHUNK // _L, vec_body, accs)

    zero = jnp.zeros((_L,), jnp.float32)
    accs = lax.fori_loop(0, nchunks, chunk_body,
                         tuple(zero for _ in range(_B * _E)))
    for b in range(_B):
        for e in range(_E):
            acc_v[b, pl.ds(e * _L, _L)] = accs[b * _E + e]
    pltpu.sync_copy(acc_v, out_hbm.at[wid])


def _sc_gate(x, W, n_sc):
    n_w = n_sc // _NW
    nchunks = n_w // _SC_CHUNK
    mesh = plsc.VectorSubcoreMesh(core_axis_name="c", subcore_axis_name="s",
                                  num_cores=_NC, num_subcores=_NS)
    body = functools.partial(_sc_body, n_w=n_w, nchunks=nchunks)
    return pl.kernel(
        body,
        out_type=jax.ShapeDtypeStruct((_NW, _B, _E * _L), jnp.float32),
        mesh=mesh,
        scratch_types=[
            pltpu.VMEM((_B, _SC_CHUNK), jnp.float32),
            pltpu.VMEM((_E, _SC_CHUNK), jnp.float32),
            pltpu.VMEM((_B, _E * _L), jnp.float32),
        ],
        name="sc_gate_partial",
    )(x, W)


def _tc_body(x_ref, w_ref, out_ref, acc_ref, *, nsteps):
    i = pl.program_id(0)

    @pl.when(i == 0)
    def _init():
        acc_ref[...] = jnp.zeros_like(acc_ref)

    acc_ref[...] += lax.dot_general(
        x_ref[...], w_ref[...],
        dimension_numbers=(((1,), (1,)), ((), ())),
        preferred_element_type=jnp.float32,
    )

    @pl.when(i == nsteps - 1)
    def _fin():
        out_ref[...] = acc_ref[...]


def _tc_gate(x, W, n_sc):
    n_tc = _N - n_sc
    nsteps = n_tc // _TC_CHUNK
    off = n_sc // _TC_CHUNK
    body = functools.partial(_tc_body, nsteps=nsteps)
    return pl.pallas_call(
        body,
        grid=(nsteps,),
        in_specs=[
            pl.BlockSpec((_B, _TC_CHUNK), lambda i: (0, i + off)),
            pl.BlockSpec((_E, _TC_CHUNK), lambda i: (0, i + off)),
        ],
        out_specs=pl.BlockSpec((_B, _E), lambda i: (0, 0)),
        out_shape=jax.ShapeDtypeStruct((_B, _E), jnp.float32),
        scratch_shapes=[pltpu.VMEM((_B, _E), jnp.float32)],
        name="tc_gate_partial",
    )(x, W)


def _combine_body(sc_ref, tc_ref, logits_ref, weights_ref, experts_ref):
    p = jnp.sum(sc_ref[...], axis=0)
    row = lax.broadcasted_iota(jnp.int32, (_E * _L, _E), 0)
    col = lax.broadcasted_iota(jnp.int32, (_E * _L, _E), 1)
    m = (row // _L == col).astype(jnp.float32)
    logits = jnp.dot(p, m, preferred_element_type=jnp.float32) + tc_ref[...]
    logits_ref[...] = logits
    mx = jnp.max(logits, axis=1, keepdims=True)
    ex = jnp.exp(logits - mx)
    w = ex / jnp.sum(ex, axis=1, keepdims=True)
    idx = lax.broadcasted_iota(jnp.int32, (_B, _E), 1)
    neg = jnp.float32(-jnp.inf)
    m1 = jnp.max(w, axis=1, keepdims=True)
    i1 = jnp.min(jnp.where(w == m1, idx, _E), axis=1, keepdims=True)
    w2 = jnp.where(idx == i1, neg, w)
    m2 = jnp.max(w2, axis=1, keepdims=True)
    i2 = jnp.min(jnp.where(w2 == m2, idx, _E), axis=1, keepdims=True)
    s = m1 + m2
    weights_ref[...] = jnp.concatenate([m1 / s, m2 / s], axis=1)
    experts_ref[...] = jnp.concatenate([i1, i2], axis=1)


def _combine(sc_part, tc_part):
    return pl.pallas_call(
        _combine_body,
        out_shape=[
            jax.ShapeDtypeStruct((_B, _E), jnp.float32),
            jax.ShapeDtypeStruct((_B, _K), jnp.float32),
            jax.ShapeDtypeStruct((_B, _K), jnp.int32),
        ],
        name="router_combine",
    )(sc_part, tc_part)


@jax.jit
def kernel(inputs, W):
    x = inputs.reshape(_B, -1)
    n_sc = _SC_FRAC_NUM * _N // 32
    sc_part = _sc_gate(x, W, n_sc)
    tc_part = _tc_gate(x, W, n_sc)
    logits, weights, experts = _combine(sc_part, tc_part)
    return (weights, experts, logits)
